# baseline (speedup 1.0000x reference)
import functools

import numpy as np

import jax
import jax.numpy as jnp
from jax import lax
from jax.experimental import pallas as pl
from jax.experimental.pallas import tpu as pltpu
from jax.experimental.pallas import tpu_sc as plsc

N = 10000
E = 320000
D = 128
L = 16
NC = 2
NS = 16
CH = 80
FH = D // NC
R = N // L
ROWS_PER_TILE = 624

EPT = E // NS
HALF = EPT // 2
HCH = HALF // CH
NPAIR = (HCH - 1) // 2


def _attn_proj_body(a_ref, x_ref, bv_ref, o_ref):
  o_ref[...] = (
      lax.dot_general(
          a_ref[...], x_ref[...], (((1,), (1,)), ((), ())),
          preferred_element_type=jnp.float32,
      )
      + bv_ref[...]
  )


def _final_body(agg_ref, w_ref, bv_ref, o_ref):
  s = jnp.concatenate([agg_ref[0], agg_ref[1]], axis=1)
  o_ref[...] = (
      lax.dot_general(
          s, w_ref[...], (((1,), (1,)), ((), ())),
          preferred_element_type=jnp.float32,
      )
      + bv_ref[...]
  )


def _gat_sc_body(src_hbm, dst_hbm, pq_hbm, x2_hbm, out_hbm,
                 p_v, q_v, den_v, idxv, exb, ridx, rows_g, rows_s,
                 den_sh, agg_sh, gsem0, gsem1, ssem0, ssem1):
  cid = lax.axis_index("c")
  sid = lax.axis_index("s")
  zf16 = jnp.zeros((L,), jnp.float32)
  iota = lax.iota(jnp.int32, L)
  gsems = (gsem0, gsem1)
  ssems = (ssem0, ssem1)

  pltpu.sync_copy(pq_hbm.at[0], p_v)
  pltpu.sync_copy(pq_hbm.at[1], q_v)

  @pl.loop(0, CH)
  def _zero_bufs(r):
    exb[0, r, :] = zf16
    exb[1, r, :] = zf16
    for f in range(FH // L):
      rows_s[0, r, pl.ds(f * L, L)] = zf16
      rows_s[1, r, pl.ds(f * L, L)] = zf16

  @pl.when(sid == 0)
  def _zero_den():
    for i in range(7):
      pltpu.sync_copy(exb.at[0], den_sh.at[pl.ds(i * CH, CH)])
    pltpu.sync_copy(exb.at[0, pl.ds(0, R - 7 * CH)],
                    den_sh.at[pl.ds(7 * CH, R - 7 * CH)])

  rbase = pl.multiple_of(sid * ROWS_PER_TILE, 8)
  for i in range(7):
    pltpu.sync_copy(rows_s.at[0], agg_sh.at[pl.ds(rbase + i * CH, CH)])

  @pl.when(sid == NS - 1)
  def _zero_agg_tail_full():
    pltpu.sync_copy(rows_s.at[0], agg_sh.at[pl.ds(rbase + 7 * CH, CH)])

  @pl.when(sid != NS - 1)
  def _zero_agg_tail_part():
    pltpu.sync_copy(
        rows_s.at[0, pl.ds(0, ROWS_PER_TILE - 7 * CH)],
        agg_sh.at[pl.ds(rbase + 7 * CH, ROWS_PER_TILE - 7 * CH)],
    )

  def _vmax_p(i, acc):
    return jnp.maximum(acc, p_v[pl.ds(i * L, L)])

  def _vmax_q(i, acc):
    return jnp.maximum(acc, q_v[pl.ds(i * L, L)])

  neg = jnp.full((L,), -jnp.inf, jnp.float32)
  mv = lax.fori_loop(0, N // L, _vmax_p, neg)
  mqv = lax.fori_loop(0, N // L, _vmax_q, neg)
  for sh in (1, 2, 4, 8):
    mv = jnp.maximum(
        mv, jnp.take_along_axis(mv, jnp.bitwise_xor(iota, sh), axis=0))
    mqv = jnp.maximum(
        mqv, jnp.take_along_axis(mqv, jnp.bitwise_xor(iota, sh), axis=0))
  c = jnp.maximum(mv + mqv, 0.0)

  plsc.subcore_barrier()

  def load_half_idx(h):
    row0 = sid * (EPT // CH) + h * HCH
    pltpu.sync_copy(src_hbm.at[pl.ds(row0, HCH)], idxv.at[0])
    pltpu.sync_copy(dst_hbm.at[pl.ds(row0, HCH)], idxv.at[1])

  def edge_vecs(k, j):
    sv = idxv[0, k, pl.ds(j * L, L)]
    dv = idxv[1, k, pl.ds(j * L, L)]
    return sv, dv

  def edge_ex(sv, dv):
    pv = plsc.load_gather(p_v, [sv])
    qv = plsc.load_gather(q_v, [dv])
    return jnp.exp(jnp.maximum(pv + qv, 0.0) - c)

  def p1_compute(k, b):
    for j in range(CH // L):
      sv, dv = edge_vecs(k, j)
      ex = edge_ex(sv, dv)
      ridx[b, pl.ds(j * L, L)] = lax.shift_right_logical(sv, 4)
      plsc.store_scatter(exb.at[b], [iota + j * L, jnp.bitwise_and(sv, L - 1)],
                         ex)

  def p1_issue(b):
    pltpu.async_copy(exb.at[b], den_sh.at[ridx.at[b]], ssems[b], add=True)

  def p1_wait(b):
    pltpu.make_async_copy(exb.at[b], den_sh.at[ridx.at[b]], ssems[b]).wait()

  def p1_unzero(k, b):
    for j in range(CH // L):
      sv = idxv[0, k, pl.ds(j * L, L)]
      plsc.store_scatter(exb.at[b], [iota + j * L, jnp.bitwise_and(sv, L - 1)],
                         zf16)

  @pl.loop(0, 2)
  def _p1_half(h):
    load_half_idx(h)

    @pl.loop(0, NPAIR)
    def _p1_pairs(g):
      for b in range(2):
        k = 2 * g + b

        @pl.when(g >= 1)
        def _drain():
          p1_wait(b)
          p1_unzero(k - 2, b)

        p1_compute(k, b)
        p1_issue(b)

    p1_wait(0)
    p1_unzero(HCH - 3, 0)
    p1_compute(HCH - 1, 0)
    p1_issue(0)
    p1_wait(1)
    p1_unzero(HCH - 2, 1)
    p1_wait(0)
    p1_unzero(HCH - 1, 0)

  plsc.subcore_barrier()
  pltpu.sync_copy(den_sh, den_v)

  def p2_issue_gather(k, b):
    pltpu.async_copy(x2_hbm.at[cid].at[idxv.at[0, k]], rows_g.at[b], gsems[b])

  def p2_wait_gather(k, b):
    pltpu.make_async_copy(x2_hbm.at[cid].at[idxv.at[0, k]], rows_g.at[b],
                          gsems[b]).wait()

  def p2_issue_scatter(k, b):
    pltpu.async_copy(rows_s.at[b], agg_sh.at[idxv.at[1, k]], ssems[b],
                     add=True)

  def p2_wait_scatter(k, b):
    pltpu.make_async_copy(rows_s.at[b], agg_sh.at[idxv.at[1, k]],
                          ssems[b]).wait()

  def p2_compute(k, b):
    for j in range(CH // L):
      sv, dv = edge_vecs(k, j)
      ex = edge_ex(sv, dv)
      den = plsc.load_gather(
          den_v,
          [lax.shift_right_logical(sv, 4), jnp.bitwise_and(sv, L - 1)],
      )
      w = ex / den
      for l in range(L):
        r = j * L + l
        wl = jnp.take_along_axis(w, jnp.full((L,), l, jnp.int32), axis=0)
        for g2 in range(FH // (2 * L)):
          v = rows_g[b, r, pl.ds(g2 * 2 * L, 2 * L)]
          a0, b0 = plsc.unpack(v, format=plsc.PackFormat.INTERLEAVED)
          rows_s[b, r, pl.ds(g2 * 2 * L, L)] = a0 * wl
          rows_s[b, r, pl.ds(g2 * 2 * L + L, L)] = b0 * wl

  @pl.loop(0, 2)
  def _p2_half(h):
    load_half_idx(h)
    p2_issue_gather(0, 0)
    p2_issue_gather(1, 1)

    @pl.loop(0, NPAIR)
    def _p2_pairs(g):
      for b in range(2):
        k = 2 * g + b
        p2_wait_gather(k, b)

        @pl.when(g >= 1)
        def _drain():
          p2_wait_scatter(k - 2, b)

        p2_compute(k, b)
        if b == 0:
          p2_issue_gather(k + 2, b)
        else:
          @pl.when(g < NPAIR - 1)
          def _issue_next():
            p2_issue_gather(k + 2, b)
        p2_issue_scatter(k, b)

    p2_wait_gather(HCH - 1, 0)
    p2_wait_scatter(HCH - 3, 0)
    p2_compute(HCH - 1, 0)
    p2_issue_scatter(HCH - 1, 0)
    p2_wait_scatter(HCH - 2, 1)
    p2_wait_scatter(HCH - 1, 0)

  plsc.subcore_barrier()

  @pl.when(sid == NS - 1)
  def _out_tail():
    pltpu.sync_copy(
        agg_sh.at[pl.ds(rbase, 640)],
        out_hbm.at[cid, pl.ds(rbase, 640)],
    )

  @pl.when(sid != NS - 1)
  def _out_main():
    pltpu.sync_copy(
        agg_sh.at[pl.ds(rbase, ROWS_PER_TILE)],
        out_hbm.at[cid, pl.ds(rbase, ROWS_PER_TILE)],
    )


_gat_sc = functools.partial(
    pl.kernel,
    out_type=jax.ShapeDtypeStruct((NC, N, FH), jnp.float32),
    mesh=plsc.VectorSubcoreMesh(core_axis_name="c", subcore_axis_name="s"),
    compiler_params=pltpu.CompilerParams(
        needs_layout_passes=False, use_tc_tiling_on_sc=False),
    scratch_types=[
        pltpu.VMEM((N,), jnp.float32),
        pltpu.VMEM((N,), jnp.float32),
        pltpu.VMEM((R, L), jnp.float32),
        pltpu.VMEM((2, HCH, CH), jnp.int32),
        pltpu.VMEM((2, CH, L), jnp.float32),
        pltpu.VMEM((2, CH), jnp.int32),
        pltpu.VMEM((2, CH, FH), jnp.bfloat16),
        pltpu.VMEM((2, CH, FH), jnp.float32),
        pltpu.VMEM_SHARED((R, L), jnp.float32),
        pltpu.VMEM_SHARED((N, FH), jnp.float32),
        pltpu.SemaphoreType.DMA,
        pltpu.SemaphoreType.DMA,
        pltpu.SemaphoreType.DMA,
        pltpu.SemaphoreType.DMA,
    ],
)(_gat_sc_body)


def kernel(x, edge_index, W_fc, b_fc, W_attn, b_attn):
  src2d = edge_index[0].astype(jnp.int32).reshape(E // CH, CH)
  dst2d = edge_index[1].astype(jnp.int32).reshape(E // CH, CH)
  a2 = W_attn.reshape(2, D)
  bv0 = jnp.concatenate([b_attn, jnp.zeros_like(b_attn)]).reshape(2, 1)

  pq = pl.pallas_call(
      _attn_proj_body,
      out_shape=jax.ShapeDtypeStruct((2, N), jnp.float32),
  )(a2, x, bv0)

  x2 = jnp.stack([x[:, :FH], x[:, FH:]])
  cols = np.empty(FH, dtype=np.int32)
  for g2 in range(FH // 32):
    for i in range(16):
      cols[g2 * 32 + 2 * i] = g2 * 32 + i
      cols[g2 * 32 + 2 * i + 1] = g2 * 32 + 16 + i
  x2b = x2[:, :, cols].astype(jnp.bfloat16)
  agg2 = _gat_sc(src2d, dst2d, pq, x2b)

  h = pl.pallas_call(
      _final_body,
      out_shape=jax.ShapeDtypeStruct((N, D), jnp.float32),
  )(agg2, W_fc, b_fc.reshape(1, D))
  return h


# speedup vs baseline: 1.0854x; 1.0854x over previous
import functools

import jax
import jax.numpy as jnp
from jax import lax
from jax.experimental import pallas as pl
from jax.experimental.pallas import tpu as pltpu
from jax.experimental.pallas import tpu_sc as plsc

N = 10000
E = 320000
D = 128
L = 16
NC = 2
NS = 16
CH = 80
FH = D // NC
R = N // L
ROWS_PER_TILE = 624

EPT = E // NS
HALF = EPT // 2
HCH = HALF // CH
NPAIR = (HCH - 1) // 2


def _attn_proj_body(a_ref, x_ref, bv_ref, pq_ref, x2_ref):
  xv = x_ref[...]
  pq_ref[...] = (
      lax.dot_general(
          a_ref[...], xv, (((1,), (1,)), ((), ())),
          preferred_element_type=jnp.float32,
      )
      + bv_ref[...]
  )
  x2_ref[0] = xv[:, :FH]
  x2_ref[1] = xv[:, FH:]


def _final_body(agg_ref, w_ref, bv_ref, o_ref):
  s = jnp.concatenate([agg_ref[0], agg_ref[1]], axis=1)
  o_ref[...] = (
      lax.dot_general(
          s, w_ref[...], (((1,), (1,)), ((), ())),
          preferred_element_type=jnp.float32,
      )
      + bv_ref[...]
  )


def _gat_sc_body(src_hbm, dst_hbm, pq_hbm, x2_hbm, out_hbm,
                 p_v, q_v, den_v, idxv, exb, ridx, rows_g, rows_s,
                 den_sh, agg_sh, gsem0, gsem1, ssem0, ssem1):
  cid = lax.axis_index("c")
  sid = lax.axis_index("s")
  zf16 = jnp.zeros((L,), jnp.float32)
  iota = lax.iota(jnp.int32, L)
  gsems = (gsem0, gsem1)
  ssems = (ssem0, ssem1)

  pltpu.sync_copy(pq_hbm.at[0], p_v)
  pltpu.sync_copy(pq_hbm.at[1], q_v)

  @pl.loop(0, CH)
  def _zero_bufs(r):
    exb[0, r, :] = zf16
    exb[1, r, :] = zf16
    for f in range(FH // L):
      rows_s[0, r, pl.ds(f * L, L)] = zf16
      rows_s[1, r, pl.ds(f * L, L)] = zf16

  @pl.when(sid == 0)
  def _zero_den():
    for i in range(7):
      pltpu.sync_copy(exb.at[0], den_sh.at[pl.ds(i * CH, CH)])
    pltpu.sync_copy(exb.at[0, pl.ds(0, R - 7 * CH)],
                    den_sh.at[pl.ds(7 * CH, R - 7 * CH)])

  rbase = pl.multiple_of(sid * ROWS_PER_TILE, 8)
  for i in range(7):
    pltpu.sync_copy(rows_s.at[0], agg_sh.at[pl.ds(rbase + i * CH, CH)])

  @pl.when(sid == NS - 1)
  def _zero_agg_tail_full():
    pltpu.sync_copy(rows_s.at[0], agg_sh.at[pl.ds(rbase + 7 * CH, CH)])

  @pl.when(sid != NS - 1)
  def _zero_agg_tail_part():
    pltpu.sync_copy(
        rows_s.at[0, pl.ds(0, ROWS_PER_TILE - 7 * CH)],
        agg_sh.at[pl.ds(rbase + 7 * CH, ROWS_PER_TILE - 7 * CH)],
    )

  def _vmax_p(i, acc):
    return jnp.maximum(acc, p_v[pl.ds(i * L, L)])

  def _vmax_q(i, acc):
    return jnp.maximum(acc, q_v[pl.ds(i * L, L)])

  neg = jnp.full((L,), -jnp.inf, jnp.float32)
  mv = lax.fori_loop(0, N // L, _vmax_p, neg)
  mqv = lax.fori_loop(0, N // L, _vmax_q, neg)
  for sh in (1, 2, 4, 8):
    mv = jnp.maximum(
        mv, jnp.take_along_axis(mv, jnp.bitwise_xor(iota, sh), axis=0))
    mqv = jnp.maximum(
        mqv, jnp.take_along_axis(mqv, jnp.bitwise_xor(iota, sh), axis=0))
  c = jnp.maximum(mv + mqv, 0.0)

  plsc.subcore_barrier()

  def load_half_idx(h):
    row0 = sid * (EPT // CH) + h * HCH
    pltpu.sync_copy(src_hbm.at[pl.ds(row0, HCH)], idxv.at[0])
    pltpu.sync_copy(dst_hbm.at[pl.ds(row0, HCH)], idxv.at[1])

  def edge_vecs(k, j):
    sv = idxv[0, k, pl.ds(j * L, L)]
    dv = idxv[1, k, pl.ds(j * L, L)]
    return sv, dv

  def edge_ex(sv, dv):
    pv = plsc.load_gather(p_v, [sv])
    qv = plsc.load_gather(q_v, [dv])
    return jnp.exp(jnp.maximum(pv + qv, 0.0) - c)

  def p1_compute(k, b):
    for j in range(CH // L):
      sv, dv = edge_vecs(k, j)
      ex = edge_ex(sv, dv)
      ridx[b, pl.ds(j * L, L)] = lax.shift_right_logical(sv, 4)
      plsc.store_scatter(exb.at[b], [iota + j * L, jnp.bitwise_and(sv, L - 1)],
                         ex)

  def p1_issue(b):
    pltpu.async_copy(exb.at[b], den_sh.at[ridx.at[b]], ssems[b], add=True)

  def p1_wait(b):
    pltpu.make_async_copy(exb.at[b], den_sh.at[ridx.at[b]], ssems[b]).wait()

  def p1_unzero(k, b):
    for j in range(CH // L):
      sv = idxv[0, k, pl.ds(j * L, L)]
      plsc.store_scatter(exb.at[b], [iota + j * L, jnp.bitwise_and(sv, L - 1)],
                         zf16)

  @pl.loop(0, 2)
  def _p1_half(h):
    load_half_idx(h)

    @pl.loop(0, NPAIR)
    def _p1_pairs(g):
      for b in range(2):
        k = 2 * g + b

        @pl.when(g >= 1)
        def _drain():
          p1_wait(b)
          p1_unzero(k - 2, b)

        p1_compute(k, b)
        p1_issue(b)

    p1_wait(0)
    p1_unzero(HCH - 3, 0)
    p1_compute(HCH - 1, 0)
    p1_issue(0)
    p1_wait(1)
    p1_unzero(HCH - 2, 1)
    p1_wait(0)
    p1_unzero(HCH - 1, 0)

  plsc.subcore_barrier()
  pltpu.sync_copy(den_sh, den_v)

  def p2_issue_gather(k, b):
    pltpu.async_copy(x2_hbm.at[cid].at[idxv.at[0, k]], rows_g.at[b], gsems[b])

  def p2_wait_gather(k, b):
    pltpu.make_async_copy(x2_hbm.at[cid].at[idxv.at[0, k]], rows_g.at[b],
                          gsems[b]).wait()

  def p2_issue_scatter(k, b):
    pltpu.async_copy(rows_s.at[b], agg_sh.at[idxv.at[1, k]], ssems[b],
                     add=True)

  def p2_wait_scatter(k, b):
    pltpu.make_async_copy(rows_s.at[b], agg_sh.at[idxv.at[1, k]],
                          ssems[b]).wait()

  def p2_compute(k, b):
    for j in range(CH // L):
      sv, dv = edge_vecs(k, j)
      ex = edge_ex(sv, dv)
      den = plsc.load_gather(
          den_v,
          [lax.shift_right_logical(sv, 4), jnp.bitwise_and(sv, L - 1)],
      )
      w = ex / den
      for l in range(L):
        r = j * L + l
        wl = jnp.take_along_axis(w, jnp.full((L,), l, jnp.int32), axis=0)
        for f in range(FH // L):
          rows_s[b, r, pl.ds(f * L, L)] = rows_g[b, r, pl.ds(f * L, L)] * wl

  @pl.loop(0, 2)
  def _p2_half(h):
    load_half_idx(h)
    p2_issue_gather(0, 0)
    p2_issue_gather(1, 1)

    @pl.loop(0, NPAIR)
    def _p2_pairs(g):
      for b in range(2):
        k = 2 * g + b
        p2_wait_gather(k, b)

        @pl.when(g >= 1)
        def _drain():
          p2_wait_scatter(k - 2, b)

        p2_compute(k, b)
        if b == 0:
          p2_issue_gather(k + 2, b)
        else:
          @pl.when(g < NPAIR - 1)
          def _issue_next():
            p2_issue_gather(k + 2, b)
        p2_issue_scatter(k, b)

    p2_wait_gather(HCH - 1, 0)
    p2_wait_scatter(HCH - 3, 0)
    p2_compute(HCH - 1, 0)
    p2_issue_scatter(HCH - 1, 0)
    p2_wait_scatter(HCH - 2, 1)
    p2_wait_scatter(HCH - 1, 0)

  plsc.subcore_barrier()

  @pl.when(sid == NS - 1)
  def _out_tail():
    pltpu.sync_copy(
        agg_sh.at[pl.ds(rbase, 640)],
        out_hbm.at[cid, pl.ds(rbase, 640)],
    )

  @pl.when(sid != NS - 1)
  def _out_main():
    pltpu.sync_copy(
        agg_sh.at[pl.ds(rbase, ROWS_PER_TILE)],
        out_hbm.at[cid, pl.ds(rbase, ROWS_PER_TILE)],
    )


_gat_sc = functools.partial(
    pl.kernel,
    out_type=jax.ShapeDtypeStruct((NC, N, FH), jnp.float32),
    mesh=plsc.VectorSubcoreMesh(core_axis_name="c", subcore_axis_name="s"),
    compiler_params=pltpu.CompilerParams(
        needs_layout_passes=False, use_tc_tiling_on_sc=False),
    scratch_types=[
        pltpu.VMEM((N,), jnp.float32),
        pltpu.VMEM((N,), jnp.float32),
        pltpu.VMEM((R, L), jnp.float32),
        pltpu.VMEM((2, HCH, CH), jnp.int32),
        pltpu.VMEM((2, CH, L), jnp.float32),
        pltpu.VMEM((2, CH), jnp.int32),
        pltpu.VMEM((2, CH, FH), jnp.float32),
        pltpu.VMEM((2, CH, FH), jnp.float32),
        pltpu.VMEM_SHARED((R, L), jnp.float32),
        pltpu.VMEM_SHARED((N, FH), jnp.float32),
        pltpu.SemaphoreType.DMA,
        pltpu.SemaphoreType.DMA,
        pltpu.SemaphoreType.DMA,
        pltpu.SemaphoreType.DMA,
    ],
)(_gat_sc_body)


def kernel(x, edge_index, W_fc, b_fc, W_attn, b_attn):
  src2d = edge_index[0].astype(jnp.int32).reshape(E // CH, CH)
  dst2d = edge_index[1].astype(jnp.int32).reshape(E // CH, CH)
  a2 = W_attn.reshape(2, D)
  bv0 = jnp.concatenate([b_attn, jnp.zeros_like(b_attn)]).reshape(2, 1)

  pq, x2 = pl.pallas_call(
      _attn_proj_body,
      out_shape=[
          jax.ShapeDtypeStruct((2, N), jnp.float32),
          jax.ShapeDtypeStruct((2, N, FH), jnp.float32),
      ],
  )(a2, x, bv0)

  agg2 = _gat_sc(src2d, dst2d, pq, x2)

  h = pl.pallas_call(
      _final_body,
      out_shape=jax.ShapeDtypeStruct((N, D), jnp.float32),
  )(agg2, W_fc, b_fc.reshape(1, D))
  return h
